# Initial kernel scaffold; baseline (speedup 1.0000x reference)
import jax
import jax.numpy as jnp
from jax.experimental import pallas as pl


def kernel(x, Wr, br, W1, W2, W3):
    raise NotImplementedError("write your pallas kernel here")


# speedup vs baseline: 1.1463x; 1.1463x over previous
import functools

import jax
import jax.numpy as jnp
from jax.experimental import pallas as pl
from jax.experimental.pallas import tpu as pltpu

S = 2048
D = 1024
F = 2816
E = 8
K = 2
M = 256
NF = 2
FC = F // NF
G = (S * K) // M + E - 1
EPAD = 128


def _router_kernel(x_ref, wr_ref, brp_ref, i1_ref, i2_ref, w1_ref, w2_ref):
    x = x_ref[...]
    logits = jnp.dot(x, wr_ref[...], preferred_element_type=jnp.float32)
    logits = logits + brp_ref[...]
    m1 = jnp.max(logits, axis=-1)
    i1 = jnp.argmax(logits, axis=-1).astype(jnp.int32)
    cols = jax.lax.broadcasted_iota(jnp.int32, logits.shape, 1)
    masked = jnp.where(cols == i1[:, None], -jnp.inf, logits)
    m2 = jnp.max(masked, axis=-1)
    i2 = jnp.argmax(masked, axis=-1).astype(jnp.int32)
    w1 = jax.nn.sigmoid(m1 - m2)
    i1_ref[...] = i1[:, None]
    i2_ref[...] = i2[:, None]
    w1_ref[...] = w1[:, None]
    w2_ref[...] = (1.0 - w1)[:, None]


def _ffn_kernel(eids_ref, nact_ref, tok_ref,
                x_ref, w_ref, W1_ref, W3_ref, W2_ref,
                out_ref,
                xg_ref, yacc_ref):
    g = pl.program_id(0)
    j = pl.program_id(1)

    @pl.when((g == 0) & (j == 0))
    def _init():
        out_ref[...] = jnp.zeros_like(out_ref)

    @pl.when(g < nact_ref[0])
    def _active():
        @pl.when(j == 0)
        def _gather():
            def body(i, _):
                t = tok_ref[g * M + i]
                xg_ref[i, :] = x_ref[t, :]
                return 0
            jax.lax.fori_loop(0, M, body, 0)

        xg = xg_ref[...]
        h1 = jnp.dot(xg, W1_ref[0], preferred_element_type=jnp.float32)
        h3 = jnp.dot(xg, W3_ref[0], preferred_element_type=jnp.float32)
        h = (h1 * jax.nn.sigmoid(h1)) * h3
        y = jnp.dot(h, W2_ref[0], preferred_element_type=jnp.float32)

        @pl.when(j == 0)
        def _set():
            yacc_ref[...] = y

        @pl.when(j > 0)
        def _acc():
            yacc_ref[...] = yacc_ref[...] + y

        @pl.when(j == NF - 1)
        def _scatter():
            yacc_ref[...] = yacc_ref[...] * w_ref[0]

            def body(i, _):
                t = tok_ref[g * M + i]
                out_ref[t, :] = out_ref[t, :] + yacc_ref[i, :]
                return 0
            jax.lax.fori_loop(0, M, body, 0)


def _dispatch(i1, i2, w1, w2):
    e_flat = jnp.concatenate([i1[:, 0], i2[:, 0]])
    t_flat = jnp.concatenate([jnp.arange(S, dtype=jnp.int32)] * 2)
    w_flat = jnp.concatenate([w1[:, 0], w2[:, 0]])
    order = jnp.argsort(e_flat)
    se = e_flat[order]
    st = t_flat[order]
    sw = w_flat[order]
    counts = jnp.bincount(e_flat, length=E)
    blocks_per = (counts + M - 1) // M
    cumb = jnp.cumsum(blocks_per)
    total_blocks = cumb[-1]
    gidx = jnp.minimum(jnp.arange(G, dtype=jnp.int32), total_blocks - 1)
    eids = jnp.searchsorted(cumb, gidx, side="right").astype(jnp.int32)
    group_start = jnp.concatenate([jnp.zeros(1, counts.dtype), jnp.cumsum(counts)[:-1]])
    pad_start = jnp.concatenate([jnp.zeros(1, cumb.dtype), cumb[:-1]]) * M
    rank = jnp.arange(S * K) - group_start[se]
    slot = (pad_start[se] + rank).astype(jnp.int32)
    tok = jnp.zeros((G * M,), jnp.int32).at[slot].set(st)
    wts = jnp.zeros((G * M,), jnp.float32).at[slot].set(sw)
    nact = total_blocks.astype(jnp.int32)[None]
    return eids, nact, tok, wts.reshape(G, M, 1)


def kernel(x, Wr, br, W1, W2, W3):
    xf = x.reshape(S, D)
    wrp = jnp.zeros((D, EPAD), jnp.float32).at[:, :E].set(Wr)
    brp = jnp.full((EPAD,), -jnp.inf, jnp.float32).at[:E].set(br)

    i1, i2, w1, w2 = pl.pallas_call(
        _router_kernel,
        out_shape=[
            jax.ShapeDtypeStruct((S, 1), jnp.int32),
            jax.ShapeDtypeStruct((S, 1), jnp.int32),
            jax.ShapeDtypeStruct((S, 1), jnp.float32),
            jax.ShapeDtypeStruct((S, 1), jnp.float32),
        ],
    )(xf, wrp, brp)

    eids, nact, tok, wts = _dispatch(i1, i2, w1, w2)

    grid_spec = pltpu.PrefetchScalarGridSpec(
        num_scalar_prefetch=3,
        grid=(G, NF),
        in_specs=[
            pl.BlockSpec((S, D), lambda g, j, eids, nact, tok: (0, 0)),
            pl.BlockSpec((1, M, 1), lambda g, j, eids, nact, tok: (g, 0, 0)),
            pl.BlockSpec((1, D, FC), lambda g, j, eids, nact, tok: (eids[g], 0, j)),
            pl.BlockSpec((1, D, FC), lambda g, j, eids, nact, tok: (eids[g], 0, j)),
            pl.BlockSpec((1, FC, D), lambda g, j, eids, nact, tok: (eids[g], j, 0)),
        ],
        out_specs=pl.BlockSpec((S, D), lambda g, j, eids, nact, tok: (0, 0)),
        scratch_shapes=[
            pltpu.VMEM((M, D), jnp.float32),
            pltpu.VMEM((M, D), jnp.float32),
        ],
    )

    out = pl.pallas_call(
        _ffn_kernel,
        grid_spec=grid_spec,
        out_shape=jax.ShapeDtypeStruct((S, D), jnp.float32),
    )(eids, nact, tok, xf, wts, W1, W3, W2)

    return out.reshape(x.shape)
